# baseline (speedup 1.0000x reference)
import functools

import numpy as np
import jax
import jax.numpy as jnp
from jax import lax
from jax.experimental import pallas as pl
from jax.experimental.pallas import tpu as pltpu
from jax.experimental.pallas import tpu_sc as plsc

_NUM_LEVELS = 16
_LEVEL_DIM = 4
_T = 2 ** 14
_BASE_RES = 16
_SCALE = float(np.exp2(np.log2(1024 / 16) / (_NUM_LEVELS - 1)))
_RES = tuple(int(np.floor(_BASE_RES * (_SCALE ** l))) for l in range(_NUM_LEVELS))
_IS_DENSE = tuple((r + 1) ** 3 <= _T for r in _RES)
_N_DENSE = sum(_IS_DENSE)
assert all(_IS_DENSE[l] == (l < _N_DENSE) for l in range(_NUM_LEVELS))
_P2 = np.uint32(2654435761)
_P3 = np.uint32(805459861)

_N = 524288
_NB = 2048
_LANES = 16


def _encode_tile_body(xT_hbm, emb_hbm, feat_hbm, tab0_v, tab1_v, tab2_v, tab3_v, x_v, out_v):
    level = lax.axis_index("s")
    half = lax.axis_index("c")
    npts = _N // 2

    res_i = jnp.int32(_RES[0])
    for l in range(1, _NUM_LEVELS):
        res_i = jnp.where(level == l, jnp.int32(_RES[l]), res_i)
    res_f = res_i.astype(jnp.float32)
    vv = res_i + 1

    pltpu.sync_copy(emb_hbm.at[level, 0], tab0_v)
    pltpu.sync_copy(emb_hbm.at[level, 1], tab1_v)
    pltpu.sync_copy(emb_hbm.at[level, 2], tab2_v)
    pltpu.sync_copy(emb_hbm.at[level, 3], tab3_v)

    def make_block_loop(hashed: bool):
        def chunk_body(i, _):
            o = i * _LANES
            px = x_v[0, pl.ds(o, _LANES)] * res_f
            py = x_v[1, pl.ds(o, _LANES)] * res_f
            pz = x_v[2, pl.ds(o, _LANES)] * res_f
            ix = px.astype(jnp.int32)
            iy = py.astype(jnp.int32)
            iz = pz.astype(jnp.int32)
            fx = px - ix.astype(jnp.float32)
            fy = py - iy.astype(jnp.float32)
            fz = pz - iz.astype(jnp.float32)
            wx = (1.0 - fx, fx)
            wy = (1.0 - fy, fy)
            wz = (1.0 - fz, fz)
            if hashed:
                hx = (ix.astype(jnp.uint32), (ix + 1).astype(jnp.uint32))
                hy = (iy.astype(jnp.uint32) * _P2,
                      (iy + 1).astype(jnp.uint32) * _P2)
                hz = (iz.astype(jnp.uint32) * _P3,
                      (iz + 1).astype(jnp.uint32) * _P3)
            else:
                dx = (ix, ix + 1)
                dy = (iy * vv, (iy + 1) * vv)
                dz = (iz * (vv * vv), (iz + 1) * (vv * vv))
            acc0 = jnp.zeros((_LANES,), jnp.float32)
            acc1 = jnp.zeros((_LANES,), jnp.float32)
            acc2 = jnp.zeros((_LANES,), jnp.float32)
            acc3 = jnp.zeros((_LANES,), jnp.float32)
            for bz in (0, 1):
                for by in (0, 1):
                    if hashed:
                        hyz = hy[by] ^ hz[bz]
                    else:
                        byz = dy[by] + dz[bz]
                    wyz = wy[by] * wz[bz]
                    for bx in (0, 1):
                        if hashed:
                            idx = ((hx[bx] ^ hyz) & np.uint32(_T - 1)
                                   ).astype(jnp.int32)
                        else:
                            idx = dx[bx] + byz
                        w = wx[bx] * wyz
                        g0 = plsc.load_gather(tab0_v, [idx])
                        g1 = plsc.load_gather(tab1_v, [idx])
                        g2 = plsc.load_gather(tab2_v, [idx])
                        g3 = plsc.load_gather(tab3_v, [idx])
                        acc0 = acc0 + w * g0
                        acc1 = acc1 + w * g1
                        acc2 = acc2 + w * g2
                        acc3 = acc3 + w * g3
            out_v[0, pl.ds(o, _LANES)] = acc0
            out_v[1, pl.ds(o, _LANES)] = acc1
            out_v[2, pl.ds(o, _LANES)] = acc2
            out_v[3, pl.ds(o, _LANES)] = acc3
            return 0

        def block_body(b, _):
            base = half * npts + b * _NB
            pltpu.sync_copy(xT_hbm.at[:, pl.ds(base, _NB)], x_v)
            lax.fori_loop(0, _NB // _LANES, chunk_body, 0, unroll=2)
            pltpu.sync_copy(
                out_v, feat_hbm.at[pl.ds(4 * level, 4), pl.ds(base, _NB)])
            return 0

        return block_body

    hashed_loop = make_block_loop(True)
    dense_loop = make_block_loop(False)
    nblocks = npts // _NB

    def _run(loop):
        def f():
            lax.fori_loop(0, nblocks, loop, 0, unroll=False)
        return f

    lax.cond(level >= _N_DENSE, _run(hashed_loop), _run(dense_loop))


def _sc_encode(xT, embeddings):
    mesh = plsc.VectorSubcoreMesh(core_axis_name="c", subcore_axis_name="s")
    kern = functools.partial(
        pl.kernel,
        mesh=mesh,
        compiler_params=pltpu.CompilerParams(needs_layout_passes=False),
        out_type=jax.ShapeDtypeStruct((4 * _NUM_LEVELS, _N), jnp.float32),
        scratch_types=[
            pltpu.VMEM((_T,), jnp.float32),
            pltpu.VMEM((_T,), jnp.float32),
            pltpu.VMEM((_T,), jnp.float32),
            pltpu.VMEM((_T,), jnp.float32),
            pltpu.VMEM((3, _NB), jnp.float32),
            pltpu.VMEM((4, _NB), jnp.float32),
        ],
    )(_encode_tile_body)
    return kern(xT, embeddings.transpose(0, 2, 1))


_BN = 4096


def _mlp_body(xT_ref, feat_ref, w0a_ref, w0b_ref, w1_ref, w2_ref, out_ref):
    xin = 2.0 * xT_ref[...] - 1.0
    h = jnp.dot(w0a_ref[...], xin,
                preferred_element_type=jnp.float32)
    h = h + jnp.dot(w0b_ref[...], feat_ref[...],
                    preferred_element_type=jnp.float32)
    h = jnp.maximum(h, 0.0)
    h = jnp.maximum(jnp.dot(w1_ref[...], h,
                            preferred_element_type=jnp.float32), 0.0)
    out_ref[...] = jnp.abs(jnp.dot(w2_ref[...], h,
                                   preferred_element_type=jnp.float32))


def _tc_mlp(xT, featT, W0, W1, W2):
    w0a = W0[:3].T
    w0b = W0[3:].T
    w1t = W1.T
    w2t = W2.T
    grid = (_N // _BN,)
    outT = pl.pallas_call(
        _mlp_body,
        grid=grid,
        in_specs=[
            pl.BlockSpec((3, _BN), lambda i: (0, i)),
            pl.BlockSpec((64, _BN), lambda i: (0, i)),
            pl.BlockSpec((64, 3), lambda i: (0, 0)),
            pl.BlockSpec((64, 64), lambda i: (0, 0)),
            pl.BlockSpec((64, 64), lambda i: (0, 0)),
            pl.BlockSpec((3, 64), lambda i: (0, 0)),
        ],
        out_specs=pl.BlockSpec((3, _BN), lambda i: (0, i)),
        out_shape=jax.ShapeDtypeStruct((3, _N), jnp.float32),
    )(xT, featT, w0a, w0b, w1t, w2t)
    return outT.T


def kernel(x, embeddings, W0, W1, W2):
    xT = x.T
    featT = _sc_encode(xT, embeddings)
    return _tc_mlp(xT, featT, W0, W1, W2)


# speedup vs baseline: 1.0459x; 1.0459x over previous
import functools

import numpy as np
import jax
import jax.numpy as jnp
from jax import lax
from jax.experimental import pallas as pl
from jax.experimental.pallas import tpu as pltpu
from jax.experimental.pallas import tpu_sc as plsc

_NUM_LEVELS = 16
_LEVEL_DIM = 4
_T = 2 ** 14
_BASE_RES = 16
_SCALE = float(np.exp2(np.log2(1024 / 16) / (_NUM_LEVELS - 1)))
_RES = tuple(int(np.floor(_BASE_RES * (_SCALE ** l))) for l in range(_NUM_LEVELS))
_IS_DENSE = tuple((r + 1) ** 3 <= _T for r in _RES)
_N_DENSE = sum(_IS_DENSE)
assert all(_IS_DENSE[l] == (l < _N_DENSE) for l in range(_NUM_LEVELS))
_P2 = np.uint32(2654435761)
_P3 = np.uint32(805459861)

_N = 524288
_NB = 2048
_LANES = 16


def _encode_tile_body(xT_hbm, emb_hbm, feat_hbm, tabA_v, tabB_v, x_v, out_v):
    level = lax.axis_index("s")
    half = lax.axis_index("c")
    npts = _N // 2

    res_i = jnp.int32(_RES[0])
    for l in range(1, _NUM_LEVELS):
        res_i = jnp.where(level == l, jnp.int32(_RES[l]), res_i)
    res_f = res_i.astype(jnp.float32)
    vv = res_i + 1

    pltpu.sync_copy(emb_hbm.at[level, 0], tabA_v)
    pltpu.sync_copy(emb_hbm.at[level, 1], tabB_v)

    def make_block_loop(hashed: bool):
        def chunk_body(i, _):
            o = i * _LANES
            px = x_v[0, pl.ds(o, _LANES)] * res_f
            py = x_v[1, pl.ds(o, _LANES)] * res_f
            pz = x_v[2, pl.ds(o, _LANES)] * res_f
            ix = px.astype(jnp.int32)
            iy = py.astype(jnp.int32)
            iz = pz.astype(jnp.int32)
            fx = px - ix.astype(jnp.float32)
            fy = py - iy.astype(jnp.float32)
            fz = pz - iz.astype(jnp.float32)
            wx = (1.0 - fx, fx)
            wy = (1.0 - fy, fy)
            wz = (1.0 - fz, fz)
            if hashed:
                hx = (ix.astype(jnp.uint32), (ix + 1).astype(jnp.uint32))
                hy = (iy.astype(jnp.uint32) * _P2,
                      (iy + 1).astype(jnp.uint32) * _P2)
                hz = (iz.astype(jnp.uint32) * _P3,
                      (iz + 1).astype(jnp.uint32) * _P3)
            else:
                dx = (ix, ix + 1)
                dy = (iy * vv, (iy + 1) * vv)
                dz = (iz * (vv * vv), (iz + 1) * (vv * vv))
            acc0 = jnp.zeros((_LANES,), jnp.float32)
            acc1 = jnp.zeros((_LANES,), jnp.float32)
            acc2 = jnp.zeros((_LANES,), jnp.float32)
            acc3 = jnp.zeros((_LANES,), jnp.float32)
            for bz in (0, 1):
                for by in (0, 1):
                    if hashed:
                        hyz = hy[by] ^ hz[bz]
                    else:
                        byz = dy[by] + dz[bz]
                    wyz = wy[by] * wz[bz]
                    for bx in (0, 1):
                        if hashed:
                            idx = ((hx[bx] ^ hyz) & np.uint32(_T - 1)
                                   ).astype(jnp.int32)
                        else:
                            idx = dx[bx] + byz
                        w = wx[bx] * wyz
                        ga = plsc.load_gather(tabA_v, [idx])
                        gb = plsc.load_gather(tabB_v, [idx])
                        g0 = lax.bitcast_convert_type(
                            jnp.left_shift(ga, 16), jnp.float32)
                        g1 = lax.bitcast_convert_type(
                            ga & jnp.int32(-65536), jnp.float32)
                        g2 = lax.bitcast_convert_type(
                            jnp.left_shift(gb, 16), jnp.float32)
                        g3 = lax.bitcast_convert_type(
                            gb & jnp.int32(-65536), jnp.float32)
                        acc0 = acc0 + w * g0
                        acc1 = acc1 + w * g1
                        acc2 = acc2 + w * g2
                        acc3 = acc3 + w * g3
            out_v[0, pl.ds(o, _LANES)] = acc0
            out_v[1, pl.ds(o, _LANES)] = acc1
            out_v[2, pl.ds(o, _LANES)] = acc2
            out_v[3, pl.ds(o, _LANES)] = acc3
            return 0

        def block_body(b, _):
            base = half * npts + b * _NB
            pltpu.sync_copy(xT_hbm.at[:, pl.ds(base, _NB)], x_v)
            lax.fori_loop(0, _NB // _LANES, chunk_body, 0, unroll=2)
            pltpu.sync_copy(
                out_v, feat_hbm.at[pl.ds(4 * level, 4), pl.ds(base, _NB)])
            return 0

        return block_body

    hashed_loop = make_block_loop(True)
    dense_loop = make_block_loop(False)
    nblocks = npts // _NB

    def _run(loop):
        def f():
            lax.fori_loop(0, nblocks, loop, 0, unroll=False)
        return f

    lax.cond(level >= _N_DENSE, _run(hashed_loop), _run(dense_loop))


def _sc_encode(xT, embeddings):
    mesh = plsc.VectorSubcoreMesh(core_axis_name="c", subcore_axis_name="s")
    kern = functools.partial(
        pl.kernel,
        mesh=mesh,
        compiler_params=pltpu.CompilerParams(needs_layout_passes=False),
        out_type=jax.ShapeDtypeStruct((4 * _NUM_LEVELS, _N), jnp.float32),
        scratch_types=[
            pltpu.VMEM((_T,), jnp.int32),
            pltpu.VMEM((_T,), jnp.int32),
            pltpu.VMEM((3, _NB), jnp.float32),
            pltpu.VMEM((4, _NB), jnp.float32),
        ],
    )(_encode_tile_body)
    eb = lax.bitcast_convert_type(
        embeddings.astype(jnp.bfloat16), jnp.uint16).astype(jnp.uint32)
    packed = eb[..., 0::2] | (eb[..., 1::2] << 16)
    packed = lax.bitcast_convert_type(
        packed.transpose(0, 2, 1), jnp.int32)
    return kern(xT, packed)


_BN = 4096


def _mlp_body(xT_ref, feat_ref, w0a_ref, w0b_ref, w1_ref, w2_ref, out_ref):
    xin = 2.0 * xT_ref[...] - 1.0
    h = jnp.dot(w0a_ref[...], xin,
                preferred_element_type=jnp.float32)
    h = h + jnp.dot(w0b_ref[...], feat_ref[...],
                    preferred_element_type=jnp.float32)
    h = jnp.maximum(h, 0.0)
    h = jnp.maximum(jnp.dot(w1_ref[...], h,
                            preferred_element_type=jnp.float32), 0.0)
    out_ref[...] = jnp.abs(jnp.dot(w2_ref[...], h,
                                   preferred_element_type=jnp.float32))


def _tc_mlp(xT, featT, W0, W1, W2):
    w0a = W0[:3].T
    w0b = W0[3:].T
    w1t = W1.T
    w2t = W2.T
    grid = (_N // _BN,)
    outT = pl.pallas_call(
        _mlp_body,
        grid=grid,
        in_specs=[
            pl.BlockSpec((3, _BN), lambda i: (0, i)),
            pl.BlockSpec((64, _BN), lambda i: (0, i)),
            pl.BlockSpec((64, 3), lambda i: (0, 0)),
            pl.BlockSpec((64, 64), lambda i: (0, 0)),
            pl.BlockSpec((64, 64), lambda i: (0, 0)),
            pl.BlockSpec((3, 64), lambda i: (0, 0)),
        ],
        out_specs=pl.BlockSpec((3, _BN), lambda i: (0, i)),
        out_shape=jax.ShapeDtypeStruct((3, _N), jnp.float32),
    )(xT, featT, w0a, w0b, w1t, w2t)
    return outT.T


def kernel(x, embeddings, W0, W1, W2):
    xT = x.T
    featT = _sc_encode(xT, embeddings)
    return _tc_mlp(xT, featT, W0, W1, W2)


# speedup vs baseline: 1.0800x; 1.0326x over previous
import functools

import numpy as np
import jax
import jax.numpy as jnp
from jax import lax
from jax.experimental import pallas as pl
from jax.experimental.pallas import tpu as pltpu
from jax.experimental.pallas import tpu_sc as plsc

_NUM_LEVELS = 16
_LEVEL_DIM = 4
_T = 2 ** 14
_BASE_RES = 16
_SCALE = float(np.exp2(np.log2(1024 / 16) / (_NUM_LEVELS - 1)))
_RES = tuple(int(np.floor(_BASE_RES * (_SCALE ** l))) for l in range(_NUM_LEVELS))
_IS_DENSE = tuple((r + 1) ** 3 <= _T for r in _RES)
_N_DENSE = sum(_IS_DENSE)
assert all(_IS_DENSE[l] == (l < _N_DENSE) for l in range(_NUM_LEVELS))
_P2 = np.uint32(2654435761)
_P3 = np.uint32(805459861)

_N = 524288
_NB = 2048
_LANES = 16


def _encode_tile_body(npts, xT_hbm, emb_hbm, feat_hbm, tabA_v, tabB_v, x_v,
                      out_v):
    level = lax.axis_index("s")
    half = lax.axis_index("c")

    res_i = jnp.int32(_RES[0])
    for l in range(1, _NUM_LEVELS):
        res_i = jnp.where(level == l, jnp.int32(_RES[l]), res_i)
    res_f = res_i.astype(jnp.float32)
    vv = res_i + 1

    pltpu.sync_copy(emb_hbm.at[level, 0], tabA_v)
    pltpu.sync_copy(emb_hbm.at[level, 1], tabB_v)

    def make_block_loop(hashed: bool):
        def chunk_body(i, _):
            o = i * _LANES
            px = x_v[0, pl.ds(o, _LANES)] * res_f
            py = x_v[1, pl.ds(o, _LANES)] * res_f
            pz = x_v[2, pl.ds(o, _LANES)] * res_f
            ix = px.astype(jnp.int32)
            iy = py.astype(jnp.int32)
            iz = pz.astype(jnp.int32)
            fx = px - ix.astype(jnp.float32)
            fy = py - iy.astype(jnp.float32)
            fz = pz - iz.astype(jnp.float32)
            wx = (1.0 - fx, fx)
            wy = (1.0 - fy, fy)
            wz = (1.0 - fz, fz)
            if hashed:
                hx = (ix.astype(jnp.uint32), (ix + 1).astype(jnp.uint32))
                hy = (iy.astype(jnp.uint32) * _P2,
                      (iy + 1).astype(jnp.uint32) * _P2)
                hz = (iz.astype(jnp.uint32) * _P3,
                      (iz + 1).astype(jnp.uint32) * _P3)
            else:
                dx = (ix, ix + 1)
                dy = (iy * vv, (iy + 1) * vv)
                dz = (iz * (vv * vv), (iz + 1) * (vv * vv))
            acc0 = jnp.zeros((_LANES,), jnp.float32)
            acc1 = jnp.zeros((_LANES,), jnp.float32)
            acc2 = jnp.zeros((_LANES,), jnp.float32)
            acc3 = jnp.zeros((_LANES,), jnp.float32)
            for bz in (0, 1):
                for by in (0, 1):
                    if hashed:
                        hyz = hy[by] ^ hz[bz]
                    else:
                        byz = dy[by] + dz[bz]
                    wyz = wy[by] * wz[bz]
                    for bx in (0, 1):
                        if hashed:
                            idx = ((hx[bx] ^ hyz) & np.uint32(_T - 1)
                                   ).astype(jnp.int32)
                        else:
                            idx = dx[bx] + byz
                        w = wx[bx] * wyz
                        ga = plsc.load_gather(tabA_v, [idx])
                        gb = plsc.load_gather(tabB_v, [idx])
                        g0 = lax.bitcast_convert_type(
                            jnp.left_shift(ga, 16), jnp.float32)
                        g1 = lax.bitcast_convert_type(
                            ga & jnp.int32(-65536), jnp.float32)
                        g2 = lax.bitcast_convert_type(
                            jnp.left_shift(gb, 16), jnp.float32)
                        g3 = lax.bitcast_convert_type(
                            gb & jnp.int32(-65536), jnp.float32)
                        acc0 = acc0 + w * g0
                        acc1 = acc1 + w * g1
                        acc2 = acc2 + w * g2
                        acc3 = acc3 + w * g3
            out_v[0, pl.ds(o, _LANES)] = acc0
            out_v[1, pl.ds(o, _LANES)] = acc1
            out_v[2, pl.ds(o, _LANES)] = acc2
            out_v[3, pl.ds(o, _LANES)] = acc3
            return 0

        def block_body(b, _):
            base = half * npts + b * _NB
            pltpu.sync_copy(xT_hbm.at[:, pl.ds(base, _NB)], x_v)
            lax.fori_loop(0, _NB // _LANES, chunk_body, 0, unroll=2)
            pltpu.sync_copy(
                out_v, feat_hbm.at[pl.ds(4 * level, 4), pl.ds(base, _NB)])
            return 0

        return block_body

    hashed_loop = make_block_loop(True)
    dense_loop = make_block_loop(False)
    nblocks = npts // _NB

    def _run(loop):
        def f():
            lax.fori_loop(0, nblocks, loop, 0, unroll=False)
        return f

    lax.cond(level >= _N_DENSE, _run(hashed_loop), _run(dense_loop))


def _sc_encode(xT_stripe, packed_emb, ns):
    mesh = plsc.VectorSubcoreMesh(core_axis_name="c", subcore_axis_name="s")
    kern = functools.partial(
        pl.kernel,
        mesh=mesh,
        compiler_params=pltpu.CompilerParams(needs_layout_passes=False),
        out_type=jax.ShapeDtypeStruct((4 * _NUM_LEVELS, ns), jnp.float32),
        scratch_types=[
            pltpu.VMEM((_T,), jnp.int32),
            pltpu.VMEM((_T,), jnp.int32),
            pltpu.VMEM((3, _NB), jnp.float32),
            pltpu.VMEM((4, _NB), jnp.float32),
        ],
    )(functools.partial(_encode_tile_body, ns // 2))
    return kern(xT_stripe, packed_emb)


def _pack_tables(embeddings):
    eb = lax.bitcast_convert_type(
        embeddings.astype(jnp.bfloat16), jnp.uint16).astype(jnp.uint32)
    packed = eb[..., 0::2] | (eb[..., 1::2] << 16)
    return lax.bitcast_convert_type(
        packed.transpose(0, 2, 1), jnp.int32)


_BN = 4096


def _mlp_body(xT_ref, feat_ref, w0a_ref, w0b_ref, w1_ref, w2_ref, out_ref):
    xin = 2.0 * xT_ref[...] - 1.0
    h = jnp.dot(w0a_ref[...], xin,
                preferred_element_type=jnp.float32)
    h = h + jnp.dot(w0b_ref[...], feat_ref[...],
                    preferred_element_type=jnp.float32)
    h = jnp.maximum(h, 0.0)
    h = jnp.maximum(jnp.dot(w1_ref[...], h,
                            preferred_element_type=jnp.float32), 0.0)
    out_ref[...] = jnp.abs(jnp.dot(w2_ref[...], h,
                                   preferred_element_type=jnp.float32))


def _tc_mlp(xT, featT, W0, W1, W2, ns):
    w0a = W0[:3].T
    w0b = W0[3:].T
    w1t = W1.T
    w2t = W2.T
    grid = (ns // _BN,)
    outT = pl.pallas_call(
        _mlp_body,
        grid=grid,
        in_specs=[
            pl.BlockSpec((3, _BN), lambda i: (0, i)),
            pl.BlockSpec((64, _BN), lambda i: (0, i)),
            pl.BlockSpec((64, 3), lambda i: (0, 0)),
            pl.BlockSpec((64, 64), lambda i: (0, 0)),
            pl.BlockSpec((64, 64), lambda i: (0, 0)),
            pl.BlockSpec((3, 64), lambda i: (0, 0)),
        ],
        out_specs=pl.BlockSpec((3, _BN), lambda i: (0, i)),
        out_shape=jax.ShapeDtypeStruct((3, ns), jnp.float32),
    )(xT, featT, w0a, w0b, w1t, w2t)
    return outT


_STRIPES = 2


def kernel(x, embeddings, W0, W1, W2):
    xT = x.T
    packed = _pack_tables(embeddings)
    ns = _N // _STRIPES
    outs = []
    for s in range(_STRIPES):
        xs = lax.slice(xT, (0, s * ns), (3, (s + 1) * ns))
        feat_s = _sc_encode(xs, packed, ns)
        outs.append(_tc_mlp(xs, feat_s, W0, W1, W2, ns))
    return jnp.concatenate(outs, axis=1).T


# speedup vs baseline: 1.2601x; 1.1668x over previous
import functools

import numpy as np
import jax
import jax.numpy as jnp
from jax import lax
from jax.experimental import pallas as pl
from jax.experimental.pallas import tpu as pltpu
from jax.experimental.pallas import tpu_sc as plsc

_NUM_LEVELS = 16
_LEVEL_DIM = 4
_T = 2 ** 14
_BASE_RES = 16
_SCALE = float(np.exp2(np.log2(1024 / 16) / (_NUM_LEVELS - 1)))
_RES = tuple(int(np.floor(_BASE_RES * (_SCALE ** l))) for l in range(_NUM_LEVELS))
_IS_DENSE = tuple((r + 1) ** 3 <= _T for r in _RES)
_N_DENSE = sum(_IS_DENSE)
assert all(_IS_DENSE[l] == (l < _N_DENSE) for l in range(_NUM_LEVELS))
_P2 = np.uint32(2654435761)
_P3 = np.uint32(805459861)

_N = 524288
_NB = 2048
_LANES = 16


def _encode_tile_body(npts, xT_hbm, emb_hbm, feat_hbm, tabA_v, tabB_v, x_v,
                      out_v):
    level = lax.axis_index("s")
    half = lax.axis_index("c")

    res_i = jnp.int32(_RES[0])
    for l in range(1, _NUM_LEVELS):
        res_i = jnp.where(level == l, jnp.int32(_RES[l]), res_i)
    res_f = res_i.astype(jnp.float32)
    vv = res_i + 1

    pltpu.sync_copy(emb_hbm.at[level, 0], tabA_v)
    pltpu.sync_copy(emb_hbm.at[level, 1], tabB_v)

    def make_block_loop(hashed: bool):
        def chunk_body(i, _):
            o = i * _LANES
            px = x_v[0, pl.ds(o, _LANES)] * res_f
            py = x_v[1, pl.ds(o, _LANES)] * res_f
            pz = x_v[2, pl.ds(o, _LANES)] * res_f
            ix = px.astype(jnp.int32)
            iy = py.astype(jnp.int32)
            iz = pz.astype(jnp.int32)
            fx = px - ix.astype(jnp.float32)
            fy = py - iy.astype(jnp.float32)
            fz = pz - iz.astype(jnp.float32)
            wx = (1.0 - fx, fx)
            wy = (1.0 - fy, fy)
            wz = (1.0 - fz, fz)
            if hashed:
                hx = (ix.astype(jnp.uint32), (ix + 1).astype(jnp.uint32))
                hy = (iy.astype(jnp.uint32) * _P2,
                      (iy + 1).astype(jnp.uint32) * _P2)
                hz = (iz.astype(jnp.uint32) * _P3,
                      (iz + 1).astype(jnp.uint32) * _P3)
            else:
                dx = (ix, ix + 1)
                dy = (iy * vv, (iy + 1) * vv)
                dz = (iz * (vv * vv), (iz + 1) * (vv * vv))
            acc0 = jnp.zeros((_LANES,), jnp.float32)
            acc1 = jnp.zeros((_LANES,), jnp.float32)
            acc2 = jnp.zeros((_LANES,), jnp.float32)
            acc3 = jnp.zeros((_LANES,), jnp.float32)
            for bz in (0, 1):
                for by in (0, 1):
                    if hashed:
                        hyz = hy[by] ^ hz[bz]
                    else:
                        byz = dy[by] + dz[bz]
                    wyz = wy[by] * wz[bz]
                    for bx in (0, 1):
                        if hashed:
                            idx = ((hx[bx] ^ hyz) & np.uint32(_T - 1)
                                   ).astype(jnp.int32)
                        else:
                            idx = dx[bx] + byz
                        w = wx[bx] * wyz
                        ga = plsc.load_gather(tabA_v, [idx])
                        gb = plsc.load_gather(tabB_v, [idx])
                        g0 = lax.bitcast_convert_type(
                            jnp.left_shift(ga, 16), jnp.float32)
                        g1 = lax.bitcast_convert_type(
                            ga & jnp.int32(-65536), jnp.float32)
                        g2 = lax.bitcast_convert_type(
                            jnp.left_shift(gb, 16), jnp.float32)
                        g3 = lax.bitcast_convert_type(
                            gb & jnp.int32(-65536), jnp.float32)
                        acc0 = acc0 + w * g0
                        acc1 = acc1 + w * g1
                        acc2 = acc2 + w * g2
                        acc3 = acc3 + w * g3
            out_v[0, pl.ds(o, _LANES)] = acc0
            out_v[1, pl.ds(o, _LANES)] = acc1
            out_v[2, pl.ds(o, _LANES)] = acc2
            out_v[3, pl.ds(o, _LANES)] = acc3
            return 0

        def block_body(b, _):
            base = half * npts + b * _NB
            pltpu.sync_copy(xT_hbm.at[:, pl.ds(base, _NB)], x_v)
            def _pl_body(i):
                chunk_body(i, 0)

            plsc.parallel_loop(0, _NB // _LANES, 1, unroll=2)(_pl_body)
            pltpu.sync_copy(
                out_v, feat_hbm.at[pl.ds(4 * level, 4), pl.ds(base, _NB)])
            return 0

        return block_body

    hashed_loop = make_block_loop(True)
    dense_loop = make_block_loop(False)
    nblocks = npts // _NB

    def _run(loop):
        def f():
            lax.fori_loop(0, nblocks, loop, 0, unroll=False)
        return f

    lax.cond(level >= _N_DENSE, _run(hashed_loop), _run(dense_loop))


def _sc_encode(xT_stripe, packed_emb, ns):
    mesh = plsc.VectorSubcoreMesh(core_axis_name="c", subcore_axis_name="s")
    kern = functools.partial(
        pl.kernel,
        mesh=mesh,
        compiler_params=pltpu.CompilerParams(needs_layout_passes=False),
        out_type=jax.ShapeDtypeStruct((4 * _NUM_LEVELS, ns), jnp.float32),
        scratch_types=[
            pltpu.VMEM((_T,), jnp.int32),
            pltpu.VMEM((_T,), jnp.int32),
            pltpu.VMEM((3, _NB), jnp.float32),
            pltpu.VMEM((4, _NB), jnp.float32),
        ],
    )(functools.partial(_encode_tile_body, ns // 2))
    return kern(xT_stripe, packed_emb)


def _pack_tables(embeddings):
    eb = lax.bitcast_convert_type(
        embeddings.astype(jnp.bfloat16), jnp.uint16).astype(jnp.uint32)
    packed = eb[..., 0::2] | (eb[..., 1::2] << 16)
    return lax.bitcast_convert_type(
        packed.transpose(0, 2, 1), jnp.int32)


_BN = 4096


def _mlp_body(xT_ref, feat_ref, w0a_ref, w0b_ref, w1_ref, w2_ref, out_ref):
    xin = 2.0 * xT_ref[...] - 1.0
    h = jnp.dot(w0a_ref[...], xin,
                preferred_element_type=jnp.float32)
    h = h + jnp.dot(w0b_ref[...], feat_ref[...],
                    preferred_element_type=jnp.float32)
    h = jnp.maximum(h, 0.0)
    h = jnp.maximum(jnp.dot(w1_ref[...], h,
                            preferred_element_type=jnp.float32), 0.0)
    out_ref[...] = jnp.abs(jnp.dot(w2_ref[...], h,
                                   preferred_element_type=jnp.float32))


def _tc_mlp(xT, featT, W0, W1, W2, ns):
    w0a = W0[:3].T
    w0b = W0[3:].T
    w1t = W1.T
    w2t = W2.T
    grid = (ns // _BN,)
    outT = pl.pallas_call(
        _mlp_body,
        grid=grid,
        in_specs=[
            pl.BlockSpec((3, _BN), lambda i: (0, i)),
            pl.BlockSpec((64, _BN), lambda i: (0, i)),
            pl.BlockSpec((64, 3), lambda i: (0, 0)),
            pl.BlockSpec((64, 64), lambda i: (0, 0)),
            pl.BlockSpec((64, 64), lambda i: (0, 0)),
            pl.BlockSpec((3, 64), lambda i: (0, 0)),
        ],
        out_specs=pl.BlockSpec((3, _BN), lambda i: (0, i)),
        out_shape=jax.ShapeDtypeStruct((3, ns), jnp.float32),
    )(xT, featT, w0a, w0b, w1t, w2t)
    return outT


_STRIPES = 2


def kernel(x, embeddings, W0, W1, W2):
    xT = x.T
    packed = _pack_tables(embeddings)
    ns = _N // _STRIPES
    outs = []
    for s in range(_STRIPES):
        xs = lax.slice(xT, (0, s * ns), (3, (s + 1) * ns))
        feat_s = _sc_encode(xs, packed, ns)
        outs.append(_tc_mlp(xs, feat_s, W0, W1, W2, ns))
    return jnp.concatenate(outs, axis=1).T
